# baseline (speedup 1.0000x reference)
import functools

import jax
import jax.numpy as jnp
from jax import lax
from jax.experimental import pallas as pl
from jax.experimental.pallas import tpu as pltpu
from jax.experimental.pallas import tpu_sc as plsc

_BATCH = 16384
_DIM = 64
_GAMMA = 12.0

_NC = 2
_NS = 16
_L = 16
_NW = _NC * _NS
_ROWS_PER_W = _BATCH // _NW
_CHUNK = 128
_NCHUNK = _ROWS_PER_W // _CHUNK


def _compute_chunk(rows_h, rows_r, rows_t, out_v, out_base):
    lane = lax.iota(jnp.int32, _L)

    def block_body(b, carry):
        acc = jnp.zeros((_L,), jnp.float32)
        for l in range(_L):
            row = b * _L + l
            psum = jnp.zeros((_L,), jnp.float32)
            for j in range(_DIM // _L):
                sl = pl.ds(j * _L, _L)
                hv = rows_h[row, sl]
                rv = rows_r[row, sl]
                tv = rows_t[row, sl]
                psum = psum + jnp.abs(hv + rv - tv)
            total = jnp.sum(psum) - _GAMMA
            acc = jnp.where(lane == l, total, acc)
        out_v[pl.ds(out_base + b * _L, _L)] = acc
        return carry

    lax.fori_loop(0, _CHUNK // _L, block_body, 0)


def _body(hidx_hbm, ridx_hbm, tidx_hbm, ent_hbm, rel_hbm, out_hbm,
          idx_h, idx_r, idx_t, rows_h, rows_r, rows_t, out_v, sem):
    wid = lax.axis_index("s") * _NC + lax.axis_index("c")
    base = wid * _ROWS_PER_W

    for c in range(_NCHUNK):
        src = pl.ds(base + c * _CHUNK, _CHUNK)
        pltpu.sync_copy(hidx_hbm.at[src], idx_h.at[c])
        pltpu.sync_copy(ridx_hbm.at[src], idx_r.at[c])
        pltpu.sync_copy(tidx_hbm.at[src], idx_t.at[c])

    for c in range(_NCHUNK):
        cp_h = pltpu.async_copy(ent_hbm.at[idx_h.at[c]], rows_h, sem)
        cp_r = pltpu.async_copy(rel_hbm.at[idx_r.at[c]], rows_r, sem)
        cp_t = pltpu.async_copy(ent_hbm.at[idx_t.at[c]], rows_t, sem)
        cp_h.wait()
        cp_r.wait()
        cp_t.wait()
        _compute_chunk(rows_h, rows_r, rows_t, out_v, c * _CHUNK)

    pltpu.sync_copy(out_v, out_hbm.at[pl.ds(base, _ROWS_PER_W)])


@functools.partial(
    pl.kernel,
    out_type=jax.ShapeDtypeStruct((_BATCH,), jnp.float32),
    scratch_types=[
        pltpu.VMEM((_NCHUNK, _CHUNK), jnp.int32),
        pltpu.VMEM((_NCHUNK, _CHUNK), jnp.int32),
        pltpu.VMEM((_NCHUNK, _CHUNK), jnp.int32),
        pltpu.VMEM((_CHUNK, _DIM), jnp.float32),
        pltpu.VMEM((_CHUNK, _DIM), jnp.float32),
        pltpu.VMEM((_CHUNK, _DIM), jnp.float32),
        pltpu.VMEM((_ROWS_PER_W,), jnp.float32),
        pltpu.SemaphoreType.DMA,
    ],
    mesh=plsc.VectorSubcoreMesh(core_axis_name="c", subcore_axis_name="s"),
    compiler_params=pltpu.CompilerParams(
        needs_layout_passes=False, use_tc_tiling_on_sc=False
    ),
)
def _transe_sc(*args):
    _body(*args)


def kernel(pos_sample, ent_embd, rel_embd):
    h_idx = pos_sample[:, 0]
    r_idx = pos_sample[:, 1]
    t_idx = pos_sample[:, 2]
    score = _transe_sc(h_idx, r_idx, t_idx, ent_embd, rel_embd)
    return score[:, None]
